# baseline (device time: 12731 ns/iter reference)
import jax
import jax.numpy as jnp
from jax import lax
from jax.experimental import pallas as pl
from jax.experimental.pallas import tpu as pltpu


def kernel(x):
    m_per, n = x.shape
    half = m_per // 2

    def body(x_ref, out_ref, send_sems, recv_sems):
        my_x = lax.axis_index("x")
        my_y = lax.axis_index("y")
        y_nbr = (my_x, 1 - my_y)
        diag = (1 - my_x, 1 - my_y)

        barrier_sem = pltpu.get_barrier_semaphore()
        for peer in (y_nbr, diag):
            pl.semaphore_signal(
                barrier_sem, inc=1, device_id=peer,
                device_id_type=pl.DeviceIdType.MESH,
            )
        pl.semaphore_wait(barrier_sem, 2)

        my_rows = my_y * m_per
        out_ref[pl.ds(my_rows, m_per), :] = x_ref[...].astype(jnp.bfloat16)

        rdma_near = pltpu.make_async_remote_copy(
            src_ref=out_ref.at[pl.ds(my_rows, half)],
            dst_ref=out_ref.at[pl.ds(my_rows, half)],
            send_sem=send_sems.at[0],
            recv_sem=recv_sems.at[0],
            device_id=y_nbr,
            device_id_type=pl.DeviceIdType.MESH,
        )
        rdma_far = pltpu.make_async_remote_copy(
            src_ref=out_ref.at[pl.ds(my_rows + half, half)],
            dst_ref=out_ref.at[pl.ds(my_rows + half, half)],
            send_sem=send_sems.at[1],
            recv_sem=recv_sems.at[1],
            device_id=diag,
            device_id_type=pl.DeviceIdType.MESH,
        )
        rdma_near.start()
        rdma_far.start()
        rdma_near.wait()
        rdma_far.wait()

    return pl.pallas_call(
        body,
        out_shape=jax.ShapeDtypeStruct((2 * m_per, n), jnp.bfloat16),
        in_specs=[pl.BlockSpec(memory_space=pltpu.VMEM)],
        out_specs=pl.BlockSpec(memory_space=pltpu.VMEM),
        scratch_shapes=[
            pltpu.SemaphoreType.DMA((2,)),
            pltpu.SemaphoreType.DMA((2,)),
        ],
        compiler_params=pltpu.CompilerParams(collective_id=0),
    )(x)
